# baseline (speedup 1.0000x reference)
import functools

import jax
import jax.numpy as jnp
from jax import lax
from jax.experimental import pallas as pl
from jax.experimental.pallas import tpu as pltpu
from jax.experimental.pallas import tpu_sc as plsc

_HIDDEN = 2048
_NUM_CORES = 2
_NUM_SUBCORES = 16
_NUM_WORKERS = _NUM_CORES * _NUM_SUBCORES
_CHUNK = 16
_NBUF = 3


def _emb_body(table_hbm, idx_hbm, out_hbm,
              idx_v, buf0, buf1, buf2, sem0, sem1, sem2):
    bufs = (buf0, buf1, buf2)
    sems = (sem0, sem1, sem2)
    wid = lax.axis_index("s") * _NUM_CORES + lax.axis_index("c")
    n_chunks = idx_hbm.shape[1]
    b_per_w = n_chunks * _CHUNK
    base = wid * b_per_w
    pltpu.sync_copy(idx_hbm.at[wid], idx_v)

    def fire_gather(c):
        p = c % _NBUF
        ids_vec = idx_v[c]
        for k in range(_CHUNK):
            row = ids_vec[k]
            pltpu.async_copy(
                table_hbm.at[pl.ds(row * _HIDDEN, _HIDDEN)],
                bufs[p].at[pl.ds(k * _HIDDEN, _HIDDEN)], sems[p])
        return pltpu.make_async_copy(
            table_hbm.at[pl.ds(0, _CHUNK * _HIDDEN)], bufs[p], sems[p])

    handles = {}
    for c in range(min(_NBUF, n_chunks)):
        handles[c] = fire_gather(c)
    for c in range(n_chunks):
        p = c % _NBUF
        handles.pop(c).wait()
        pltpu.sync_copy(
            bufs[p],
            out_hbm.at[pl.ds((base + c * _CHUNK) * _HIDDEN,
                             _CHUNK * _HIDDEN)])
        nxt = c + _NBUF
        if nxt < n_chunks:
            handles[nxt] = fire_gather(nxt)


def kernel(input_ids, embed_in_weight):
    out_shape = input_ids.shape + (_HIDDEN,)
    flat = input_ids.reshape(-1).astype(jnp.int32)
    total = flat.shape[0]
    b_per_w = total // _NUM_WORKERS
    n_chunks = b_per_w // _CHUNK
    idx3 = flat.reshape(_NUM_WORKERS, n_chunks, _CHUNK)
    table_1d = embed_in_weight.reshape(-1)
    mesh = plsc.VectorSubcoreMesh(core_axis_name="c", subcore_axis_name="s")
    run = functools.partial(
        pl.kernel,
        mesh=mesh,
        out_type=jax.ShapeDtypeStruct((total * _HIDDEN,), jnp.float32),
        scratch_types=(
            [pltpu.VMEM((n_chunks, _CHUNK), jnp.int32)]
            + [pltpu.VMEM((_CHUNK * _HIDDEN,), jnp.float32)] * _NBUF
            + [pltpu.SemaphoreType.DMA] * _NBUF
        ),
    )(_emb_body)
    out = run(table_1d, idx3)
    return out.reshape(out_shape)


# speedup vs baseline: 8.2175x; 8.2175x over previous
import functools

import jax
import jax.numpy as jnp
from jax import lax
from jax.experimental import pallas as pl
from jax.experimental.pallas import tpu as pltpu
from jax.experimental.pallas import tpu_sc as plsc

_HIDDEN = 2048
_NUM_CORES = 2
_NUM_SUBCORES = 16
_NUM_WORKERS = _NUM_CORES * _NUM_SUBCORES
_CHUNK = 8
_NBUF = 3
_SBUF = 2


def _emb_body(table_hbm, idx_hbm, out_hbm, idx_v, shared, *rest):
    bufs = rest[:_NBUF]
    gsems = rest[_NBUF:2 * _NBUF]
    xsems = rest[2 * _NBUF:2 * _NBUF + _SBUF]
    wsems = rest[2 * _NBUF + _SBUF:2 * _NBUF + 2 * _SBUF]
    sid = lax.axis_index("s")
    wid = sid * _NUM_CORES + lax.axis_index("c")
    n_chunks = idx_hbm.shape[1]
    b_per_w = n_chunks * _CHUNK
    base = wid * b_per_w
    pltpu.sync_copy(idx_hbm.at[wid], idx_v)

    def fire_g(c):
        p = c % _NBUF
        return pltpu.async_copy(table_hbm.at[idx_v.at[c]], bufs[p], gsems[p])

    def fire_x(c):
        p, q = c % _NBUF, c % _SBUF
        return pltpu.async_copy(bufs[p], shared.at[sid, q], xsems[q])

    def fire_w(c):
        q = c % _SBUF
        return pltpu.async_copy(
            shared.at[sid, q],
            out_hbm.at[pl.ds(base + c * _CHUNK, _CHUNK)], wsems[q])

    gh, xh, wh = {}, {}, {}
    for c in range(min(_NBUF, n_chunks)):
        gh[c] = fire_g(c)
    for c in range(n_chunks):
        gh.pop(c).wait()
        if c - _SBUF in wh:
            wh.pop(c - _SBUF).wait()
        xh[c] = fire_x(c)
        prev = c - 1
        if prev in xh:
            xh.pop(prev).wait()
            wh[prev] = fire_w(prev)
            if prev + _NBUF < n_chunks:
                gh[prev + _NBUF] = fire_g(prev + _NBUF)
    last = n_chunks - 1
    if last in xh:
        xh.pop(last).wait()
        wh[last] = fire_w(last)
    for c in sorted(wh):
        wh.pop(c).wait()


def kernel(input_ids, embed_in_weight):
    out_shape = input_ids.shape + (_HIDDEN,)
    flat = input_ids.reshape(-1).astype(jnp.int32)
    total = flat.shape[0]
    b_per_w = total // _NUM_WORKERS
    n_chunks = b_per_w // _CHUNK
    idx3 = flat.reshape(_NUM_WORKERS, n_chunks, _CHUNK)
    mesh = plsc.VectorSubcoreMesh(core_axis_name="c", subcore_axis_name="s")
    run = functools.partial(
        pl.kernel,
        mesh=mesh,
        out_type=jax.ShapeDtypeStruct((total, _HIDDEN), jnp.float32),
        scratch_types=(
            [pltpu.VMEM((n_chunks, _CHUNK), jnp.int32),
             pltpu.VMEM_SHARED((_NUM_SUBCORES, _SBUF, _CHUNK, _HIDDEN),
                               jnp.float32)]
            + [pltpu.VMEM((_CHUNK, _HIDDEN), jnp.float32)] * _NBUF
            + [pltpu.SemaphoreType.DMA] * (_NBUF + 2 * _SBUF)
        ),
    )(_emb_body)
    out = run(embed_in_weight, idx3)
    return out.reshape(out_shape)


# speedup vs baseline: 8.4833x; 1.0324x over previous
import functools

import jax
import jax.numpy as jnp
from jax import lax
from jax.experimental import pallas as pl
from jax.experimental.pallas import tpu as pltpu
from jax.experimental.pallas import tpu_sc as plsc

_HIDDEN = 2048
_NUM_CORES = 2
_NUM_SUBCORES = 16
_NUM_WORKERS = _NUM_CORES * _NUM_SUBCORES
_CHUNK = 16
_NBUF = 3


def _emb_body(table_hbm, idx_hbm, out_hbm,
              idx_v, buf0, buf1, buf2, sem0, sem1, sem2):
    bufs = (buf0, buf1, buf2)
    sems = (sem0, sem1, sem2)
    wid = lax.axis_index("s") * _NUM_CORES + lax.axis_index("c")
    n_chunks = idx_hbm.shape[1]
    b_per_w = n_chunks * _CHUNK
    base = wid * b_per_w
    pltpu.sync_copy(idx_hbm.at[wid], idx_v)
    handles = {}
    for c in range(min(_NBUF, n_chunks)):
        handles[c] = pltpu.async_copy(
            table_hbm.at[idx_v.at[c]], bufs[c % _NBUF], sems[c % _NBUF])
    for c in range(n_chunks):
        p = c % _NBUF
        handles.pop(c).wait()
        pltpu.sync_copy(bufs[p], out_hbm.at[pl.ds(base + c * _CHUNK, _CHUNK)])
        nxt = c + _NBUF
        if nxt < n_chunks:
            handles[nxt] = pltpu.async_copy(
                table_hbm.at[idx_v.at[nxt]], bufs[p], sems[p])


def kernel(input_ids, embed_in_weight):
    out_shape = input_ids.shape + (_HIDDEN,)
    flat = input_ids.reshape(-1).astype(jnp.int32)
    total = flat.shape[0]
    b_per_w = total // _NUM_WORKERS
    n_chunks = b_per_w // _CHUNK
    idx3 = flat.reshape(_NUM_WORKERS, n_chunks, _CHUNK)
    mesh = plsc.VectorSubcoreMesh(core_axis_name="c", subcore_axis_name="s")
    run = functools.partial(
        pl.kernel,
        mesh=mesh,
        out_type=jax.ShapeDtypeStruct((total, _HIDDEN), jnp.float32),
        scratch_types=(
            [pltpu.VMEM((n_chunks, _CHUNK), jnp.int32)]
            + [pltpu.VMEM((_CHUNK, _HIDDEN), jnp.float32)] * _NBUF
            + [pltpu.SemaphoreType.DMA] * _NBUF
        ),
    )(_emb_body)
    out = run(embed_in_weight, idx3)
    return out.reshape(out_shape)
